# baseline (speedup 1.0000x reference)
import functools

import jax
import jax.numpy as jnp
from jax.experimental import pallas as pl
from jax.experimental.pallas import tpu as pltpu

_BM = 512


def _mm_kernel(x_ref, w_ref, m_ref, b_ref, o_ref):
    xb = x_ref[...].astype(jnp.bfloat16)
    acc = jax.lax.dot_general(
        xb, w_ref[...],
        (((1,), (1,)), ((), ())),
        preferred_element_type=jnp.float32,
    )
    o_ref[...] = (acc + b_ref[...]) * m_ref[...]


@functools.partial(jax.jit, static_argnames=())
def kernel(input, data_mask, W, b):
    B, L, D = input.shape
    M = B * L
    x2 = input.reshape(M, D)
    maskf = data_mask.reshape(M, 1).astype(jnp.float32)
    wb = W.astype(jnp.bfloat16)
    b2 = b.reshape(1, D)

    out = pl.pallas_call(
        _mm_kernel,
        grid=(M // _BM,),
        in_specs=[
            pl.BlockSpec((_BM, D), lambda i: (i, 0)),
            pl.BlockSpec((D, D), lambda i: (0, 0)),
            pl.BlockSpec((_BM, 1), lambda i: (i, 0)),
            pl.BlockSpec((1, D), lambda i: (0, 0)),
        ],
        out_specs=pl.BlockSpec((_BM, D), lambda i: (i, 0)),
        out_shape=jax.ShapeDtypeStruct((M, D), jnp.float32),
        compiler_params=pltpu.CompilerParams(
            dimension_semantics=("parallel",),
        ),
    )(x2, wb, maskf, b2)
    return out.reshape(B, L, D)
